# baseline (speedup 1.0000x reference)
import functools

import jax
import jax.numpy as jnp
import numpy as np
from jax import lax
from jax.experimental import pallas as pl
from jax.experimental.pallas import tpu as pltpu
from jax.experimental.pallas import tpu_sc as plsc

_FIELD_DIMS = [100000] * 26
_F = 26
_D = 16
_T = 4
_B = 4096
_BF = _B * _F
_TOTAL = sum(_FIELD_DIMS)
_NLINES = _TOTAL // 8

_info = plsc.get_sparse_core_info()
_NC, _NS = _info.num_cores, _info.num_subcores
_NW = _NC * _NS
_NPW = _BF // _NW
_BW = _B // _NW

_NBLK = _TOTAL // 128 + 1
_BPW = (_NBLK + _NW - 1) // _NW
_NL = _NBLK * 16


@functools.partial(
    pl.kernel,
    out_type=jax.ShapeDtypeStruct((_NL, 128), jnp.float32),
    mesh=plsc.VectorSubcoreMesh(core_axis_name="c", subcore_axis_name="s"),
    compiler_params=pltpu.CompilerParams(needs_layout_passes=False),
    scratch_types=[
        pltpu.VMEM((_D, 128), jnp.float32),
        pltpu.VMEM((_D, 128), jnp.float32),
        pltpu.VMEM((_D, 128), jnp.float32),
        pltpu.VMEM((_D, 128), jnp.float32),
        pltpu.SemaphoreType.DMA,
        pltpu.SemaphoreType.DMA,
    ],
)
def _sc_retile(e2T_hbm, out_hbm, bufA, bufB, outA, outB, sA, sB):
    wid = lax.axis_index("s") * _NC + lax.axis_index("c")
    base = wid * _BPW
    iota = lax.iota(jnp.int32, 16)
    last = _NBLK - 1

    def load(cb, buf, sem):
        return pltpu.async_copy(e2T_hbm.at[:, pl.ds(cb * 128, 128)], buf, sem)

    def emit(cb, buf, ob):
        for l in range(_D):
            for u in range(8):
                col = jnp.full((16,), l * 8 + u, jnp.int32)
                ob[l, pl.ds(u * 16, 16)] = plsc.load_gather(buf, [iota, col])
        pltpu.sync_copy(ob, out_hbm.at[pl.ds(cb * 16, 16), :])

    def body(k, carry):
        cb0 = jnp.minimum(base + 2 * k, last)
        cb1 = jnp.minimum(base + 2 * k + 1, last)
        cA = load(cb0, bufA, sA)
        cB = load(cb1, bufB, sB)
        cA.wait()
        emit(cb0, bufA, outA)
        cB.wait()
        emit(cb1, bufB, outB)
        return carry

    lax.fori_loop(0, (_BPW + 1) // 2, body, 0)


@functools.partial(
    pl.kernel,
    out_type=[
        jax.ShapeDtypeStruct((_F * _D, _B), jnp.float32),
        jax.ShapeDtypeStruct((_B,), jnp.float32),
    ],
    mesh=plsc.VectorSubcoreMesh(core_axis_name="c", subcore_axis_name="s"),
    compiler_params=pltpu.CompilerParams(needs_layout_passes=False),
    scratch_types=[
        pltpu.VMEM((_NPW,), jnp.int32),
        pltpu.VMEM((_NPW,), jnp.int32),
        pltpu.VMEM((_NPW,), jnp.int32),
        pltpu.VMEM((_NPW,), jnp.int32),
        pltpu.VMEM((_BW, 128), jnp.float32),
        pltpu.VMEM((_BW, 128), jnp.float32),
        pltpu.VMEM((_BW, 128), jnp.float32),
        pltpu.VMEM((_BW, 128), jnp.float32),
        pltpu.VMEM((_D, _BW), jnp.float32),
        pltpu.VMEM((_BW,), jnp.float32),
        pltpu.SemaphoreType.DMA,
        pltpu.SemaphoreType.DMA,
        pltpu.SemaphoreType.DMA,
        pltpu.SemaphoreType.DMA,
    ],
)
def _sc_gather(l2_hbm, c2_hbm, l1_hbm, c1_hbm, e2l_hbm, e1l_hbm, eT_out, lin_out,
               l2v, c2v, l1v, c1v, bufA2, bufB2, bufA1, bufB1, out_f, lin_acc,
               sA, sB, sA1, sB1):
    wid = lax.axis_index("s") * _NC + lax.axis_index("c")
    base = wid * _NPW
    bcol = wid * _BW
    iota = lax.iota(jnp.int32, 16)

    pltpu.sync_copy(l2_hbm.at[pl.ds(base, _NPW)], l2v)
    pltpu.sync_copy(c2_hbm.at[pl.ds(base, _NPW)], c2v)
    pltpu.sync_copy(l1_hbm.at[pl.ds(base, _NPW)], l1v)
    pltpu.sync_copy(c1_hbm.at[pl.ds(base, _NPW)], c1v)

    zero16 = jnp.zeros((16,), jnp.float32)
    for g in range(_BW // 16):
        lin_acc[pl.ds(g * 16, 16)] = zero16

    def start(f, b2, b1, s2, s1):
        c2 = pltpu.async_copy(e2l_hbm.at[l2v.at[pl.ds(f * _BW, _BW)]], b2, s2)
        c1 = pltpu.async_copy(e1l_hbm.at[l1v.at[pl.ds(f * _BW, _BW)]], b1, s1)
        return c2, c1

    def extract(f, b2, b1):
        for g in range(_BW // 16):
            rows = iota + g * 16
            sel = f * _BW + g * 16 + iota
            r2 = plsc.load_gather(c2v, [sel])
            r1 = plsc.load_gather(c1v, [sel])
            for d in range(_D):
                v = plsc.load_gather(b2, [rows, r2 + d])
                out_f[d, pl.ds(g * 16, 16)] = v
            v1 = plsc.load_gather(b1, [rows, r1])
            lin_acc[pl.ds(g * 16, 16)] = lin_acc[pl.ds(g * 16, 16)] + v1
        pltpu.sync_copy(out_f, eT_out.at[pl.ds(f * _D, _D), pl.ds(bcol, _BW)])

    def body(k, carry):
        f0 = 2 * k
        f1 = 2 * k + 1
        cA2, cA1 = start(f0, bufA2, bufA1, sA, sA1)
        cB2, cB1 = start(f1, bufB2, bufB1, sB, sB1)
        cA2.wait()
        cA1.wait()
        extract(f0, bufA2, bufA1)
        cB2.wait()
        cB1.wait()
        extract(f1, bufB2, bufB1)
        return carry

    lax.fori_loop(0, _F // 2, body, 0)
    pltpu.sync_copy(lin_acc, lin_out.at[pl.ds(bcol, _BW)])


_BT = 128
_PAIRS = _F * (_F - 1) // 2


def _tc_body(eT_ref, lin_ref, const_ref, out_ref):
    eT = eT_ref[...]
    C = const_ref[...]

    wts = [C[_D * t:_D * (t + 1), :] for t in range(_T + 1)]
    u_parts = [[] for _ in range(_T + 1)]
    for i in range(_F - 1):
        cnt = _F - 1 - i
        left = eT[_D * i:_D * (i + 1), :]
        right = eT[_D * (i + 1):, :]
        lrep = jnp.concatenate([left] * cnt, axis=0)
        pr3 = (lrep * right).reshape(cnt, _D, _BT)
        for t in range(_T + 1):
            u_parts[t].append(jnp.sum(pr3 * wts[t][None, :, :], axis=1))
    us = [jnp.concatenate(u_parts[t], axis=0) for t in range(_T + 1)]

    score = jnp.zeros((_PAIRS, _BT), jnp.float32)
    for t in range(_T):
        b1_t = C[80 + t:81 + t, :]
        w2_t = C[84 + t:85 + t, :]
        score = score + w2_t * jnp.maximum(us[t] + b1_t, 0.0)

    m = jnp.max(score, axis=0, keepdims=True)
    ex = jnp.exp(score - m)
    z = jnp.sum(ex, axis=0, keepdims=True)
    numer = jnp.sum(ex * us[_T], axis=0, keepdims=True)
    attr_part = numer / z

    lin = lin_ref[...]
    w0v = C[88:89, :]
    logit = w0v + lin + attr_part
    out = 1.0 / (1.0 + jnp.exp(-logit))
    out_ref[...] = jnp.broadcast_to(out, (8, _BT))


def _tc_compute(eT, lin2d, const):
    grid = _B // _BT
    return pl.pallas_call(
        _tc_body,
        grid=(grid,),
        in_specs=[
            pl.BlockSpec((_F * _D, _BT), lambda i: (0, i)),
            pl.BlockSpec((1, _BT), lambda i: (0, i)),
            pl.BlockSpec((96, _BT), lambda i: (0, 0)),
        ],
        out_specs=pl.BlockSpec((8, _BT), lambda i: (0, i)),
        out_shape=jax.ShapeDtypeStruct((8, _B), jnp.float32),
    )(eT, lin2d, const)


def kernel(x, emb1, emb2, w0, p, W1, b1, W2):
    offsets = jnp.asarray(np.cumsum([0] + _FIELD_DIMS[:-1]), dtype=x.dtype)
    idxm = x + offsets[None, :]
    idx_w = idxm.T.reshape(_F, _NW, _BW).transpose(1, 0, 2).reshape(_NW * _NPW)
    lines2 = idx_w // 8
    cols2 = (idx_w % 8) * _D
    lines1 = idx_w // 128
    cols1 = idx_w % 128

    e2lines = _sc_retile(emb2.T)
    _L1 = _TOTAL // 128 + 1
    e1lines = jnp.pad(emb1, ((0, _L1 * 128 - _TOTAL), (0, 0))).reshape(_L1, 128)

    eT, lin = _sc_gather(lines2, cols2, lines1, cols1, e2lines, e1lines)

    W5 = jnp.concatenate([W1, p[:, None]], axis=1)
    top = jnp.repeat(W5.T.reshape(5 * _D, 1), _BT, axis=1)
    sc9 = jnp.concatenate([b1, W2[:, 0], w0, jnp.zeros((7,), jnp.float32)])
    bot = jnp.repeat(sc9.reshape(16, 1), _BT, axis=1)
    const = jnp.concatenate([top, bot], axis=0)

    o8 = _tc_compute(eT, lin.reshape(1, _B), const)
    return o8[0].reshape(_B, 1)


# speedup vs baseline: 2.1210x; 2.1210x over previous
import functools

import jax
import jax.numpy as jnp
import numpy as np
from jax import lax
from jax.experimental import pallas as pl
from jax.experimental.pallas import tpu as pltpu
from jax.experimental.pallas import tpu_sc as plsc

_FIELD_DIMS = [100000] * 26
_F = 26
_D = 16
_T = 4
_B = 4096
_BF = _B * _F
_TOTAL = sum(_FIELD_DIMS)
_NLINES = _TOTAL // 8

_info = plsc.get_sparse_core_info()
_NC, _NS = _info.num_cores, _info.num_subcores
_NW = _NC * _NS
_NPW = _BF // _NW
_BW = _B // _NW

_NBLK = _TOTAL // 128 + 1
_BPW = (_NBLK + _NW - 1) // _NW
_NL = _NBLK * 16


@functools.partial(
    pl.kernel,
    out_type=jax.ShapeDtypeStruct((_NL, 128), jnp.float32),
    mesh=plsc.VectorSubcoreMesh(core_axis_name="c", subcore_axis_name="s"),
    compiler_params=pltpu.CompilerParams(needs_layout_passes=False),
    scratch_types=[
        pltpu.VMEM((_D, 128), jnp.float32),
        pltpu.VMEM((_D, 128), jnp.float32),
        pltpu.VMEM((_D, 128), jnp.float32),
        pltpu.VMEM((_D, 128), jnp.float32),
        pltpu.SemaphoreType.DMA,
        pltpu.SemaphoreType.DMA,
    ],
)
def _sc_retile(e2T_hbm, out_hbm, bufA, bufB, outA, outB, sA, sB):
    wid = lax.axis_index("s") * _NC + lax.axis_index("c")
    base = wid * _BPW
    iota = lax.iota(jnp.int32, 16)
    last = _NBLK - 1

    rowv0 = iota >> 3
    colv = [((iota & 7) << 4) + d for d in range(_D)]

    def load(cb, buf, sem):
        return pltpu.async_copy(e2T_hbm.at[:, pl.ds(cb * 128, 128)], buf, sem)

    def emit(cb, buf, ob):
        for c in range(8):
            rowv = rowv0 + 2 * c
            vals = [buf[d, pl.ds(c * 16, 16)] for d in range(_D)]
            for d in range(_D):
                plsc.store_scatter(ob, [rowv, colv[d]], vals[d])
        pltpu.sync_copy(ob, out_hbm.at[pl.ds(cb * 16, 16), :])

    def body(k, carry):
        cb0 = jnp.minimum(base + 2 * k, last)
        cb1 = jnp.minimum(base + 2 * k + 1, last)
        cA = load(cb0, bufA, sA)
        cB = load(cb1, bufB, sB)
        cA.wait()
        emit(cb0, bufA, outA)
        cB.wait()
        emit(cb1, bufB, outB)
        return carry

    lax.fori_loop(0, (_BPW + 1) // 2, body, 0)


@functools.partial(
    pl.kernel,
    out_type=[
        jax.ShapeDtypeStruct((_F * _D, _B), jnp.float32),
        jax.ShapeDtypeStruct((_B,), jnp.float32),
    ],
    mesh=plsc.VectorSubcoreMesh(core_axis_name="c", subcore_axis_name="s"),
    compiler_params=pltpu.CompilerParams(needs_layout_passes=False),
    scratch_types=[
        pltpu.VMEM((_NPW,), jnp.int32),
        pltpu.VMEM((_NPW,), jnp.int32),
        pltpu.VMEM((_NPW,), jnp.int32),
        pltpu.VMEM((_NPW,), jnp.int32),
        pltpu.VMEM((_BW, 128), jnp.float32),
        pltpu.VMEM((_BW, 128), jnp.float32),
        pltpu.VMEM((_BW, 128), jnp.float32),
        pltpu.VMEM((_BW, 128), jnp.float32),
        pltpu.VMEM((_D, _BW), jnp.float32),
        pltpu.VMEM((_BW,), jnp.float32),
        pltpu.SemaphoreType.DMA,
        pltpu.SemaphoreType.DMA,
        pltpu.SemaphoreType.DMA,
        pltpu.SemaphoreType.DMA,
    ],
)
def _sc_gather(l2_hbm, c2_hbm, l1_hbm, c1_hbm, e2l_hbm, e1l_hbm, eT_out, lin_out,
               l2v, c2v, l1v, c1v, bufA2, bufB2, bufA1, bufB1, out_f, lin_acc,
               sA, sB, sA1, sB1):
    wid = lax.axis_index("s") * _NC + lax.axis_index("c")
    base = wid * _NPW
    bcol = wid * _BW
    iota = lax.iota(jnp.int32, 16)

    pltpu.sync_copy(l2_hbm.at[pl.ds(base, _NPW)], l2v)
    pltpu.sync_copy(c2_hbm.at[pl.ds(base, _NPW)], c2v)
    pltpu.sync_copy(l1_hbm.at[pl.ds(base, _NPW)], l1v)
    pltpu.sync_copy(c1_hbm.at[pl.ds(base, _NPW)], c1v)

    zero16 = jnp.zeros((16,), jnp.float32)
    for g in range(_BW // 16):
        lin_acc[pl.ds(g * 16, 16)] = zero16

    def start(f, b2, b1, s2, s1):
        c2 = pltpu.async_copy(e2l_hbm.at[l2v.at[pl.ds(f * _BW, _BW)]], b2, s2)
        c1 = pltpu.async_copy(e1l_hbm.at[l1v.at[pl.ds(f * _BW, _BW)]], b1, s1)
        return c2, c1

    def extract(f, b2, b1):
        for g in range(_BW // 16):
            rows = iota + g * 16
            sel = f * _BW + g * 16 + iota
            r2 = plsc.load_gather(c2v, [sel])
            r1 = plsc.load_gather(c1v, [sel])
            for d in range(_D):
                v = plsc.load_gather(b2, [rows, r2 + d])
                out_f[d, pl.ds(g * 16, 16)] = v
            v1 = plsc.load_gather(b1, [rows, r1])
            lin_acc[pl.ds(g * 16, 16)] = lin_acc[pl.ds(g * 16, 16)] + v1
        pltpu.sync_copy(out_f, eT_out.at[pl.ds(f * _D, _D), pl.ds(bcol, _BW)])

    def body(k, carry):
        f0 = 2 * k
        f1 = 2 * k + 1
        cA2, cA1 = start(f0, bufA2, bufA1, sA, sA1)
        cB2, cB1 = start(f1, bufB2, bufB1, sB, sB1)
        cA2.wait()
        cA1.wait()
        extract(f0, bufA2, bufA1)
        cB2.wait()
        cB1.wait()
        extract(f1, bufB2, bufB1)
        return carry

    lax.fori_loop(0, _F // 2, body, 0)
    pltpu.sync_copy(lin_acc, lin_out.at[pl.ds(bcol, _BW)])


_BT = 128
_PAIRS = _F * (_F - 1) // 2


def _tc_body(eT_ref, lin_ref, const_ref, out_ref):
    eT = eT_ref[...]
    C = const_ref[...]

    wts = [C[_D * t:_D * (t + 1), :] for t in range(_T + 1)]
    u_parts = [[] for _ in range(_T + 1)]
    for i in range(_F - 1):
        cnt = _F - 1 - i
        left = eT[_D * i:_D * (i + 1), :]
        right = eT[_D * (i + 1):, :]
        lrep = jnp.concatenate([left] * cnt, axis=0)
        pr3 = (lrep * right).reshape(cnt, _D, _BT)
        for t in range(_T + 1):
            u_parts[t].append(jnp.sum(pr3 * wts[t][None, :, :], axis=1))
    us = [jnp.concatenate(u_parts[t], axis=0) for t in range(_T + 1)]

    score = jnp.zeros((_PAIRS, _BT), jnp.float32)
    for t in range(_T):
        b1_t = C[80 + t:81 + t, :]
        w2_t = C[84 + t:85 + t, :]
        score = score + w2_t * jnp.maximum(us[t] + b1_t, 0.0)

    m = jnp.max(score, axis=0, keepdims=True)
    ex = jnp.exp(score - m)
    z = jnp.sum(ex, axis=0, keepdims=True)
    numer = jnp.sum(ex * us[_T], axis=0, keepdims=True)
    attr_part = numer / z

    lin = lin_ref[...]
    w0v = C[88:89, :]
    logit = w0v + lin + attr_part
    out = 1.0 / (1.0 + jnp.exp(-logit))
    out_ref[...] = jnp.broadcast_to(out, (8, _BT))


def _tc_compute(eT, lin2d, const):
    grid = _B // _BT
    return pl.pallas_call(
        _tc_body,
        grid=(grid,),
        in_specs=[
            pl.BlockSpec((_F * _D, _BT), lambda i: (0, i)),
            pl.BlockSpec((1, _BT), lambda i: (0, i)),
            pl.BlockSpec((96, _BT), lambda i: (0, 0)),
        ],
        out_specs=pl.BlockSpec((8, _BT), lambda i: (0, i)),
        out_shape=jax.ShapeDtypeStruct((8, _B), jnp.float32),
    )(eT, lin2d, const)


def kernel(x, emb1, emb2, w0, p, W1, b1, W2):
    offsets = jnp.asarray(np.cumsum([0] + _FIELD_DIMS[:-1]), dtype=x.dtype)
    idxm = x + offsets[None, :]
    idx_w = idxm.T.reshape(_F, _NW, _BW).transpose(1, 0, 2).reshape(_NW * _NPW)
    lines2 = idx_w // 8
    cols2 = (idx_w % 8) * _D
    lines1 = idx_w // 128
    cols1 = idx_w % 128

    e2lines = _sc_retile(emb2.T)
    _L1 = _TOTAL // 128 + 1
    e1lines = jnp.pad(emb1, ((0, _L1 * 128 - _TOTAL), (0, 0))).reshape(_L1, 128)

    eT, lin = _sc_gather(lines2, cols2, lines1, cols1, e2lines, e1lines)

    W5 = jnp.concatenate([W1, p[:, None]], axis=1)
    top = jnp.repeat(W5.T.reshape(5 * _D, 1), _BT, axis=1)
    sc9 = jnp.concatenate([b1, W2[:, 0], w0, jnp.zeros((7,), jnp.float32)])
    bot = jnp.repeat(sc9.reshape(16, 1), _BT, axis=1)
    const = jnp.concatenate([top, bot], axis=0)

    o8 = _tc_compute(eT, lin.reshape(1, _B), const)
    return o8[0].reshape(_B, 1)


# speedup vs baseline: 2.2479x; 1.0598x over previous
import functools

import jax
import jax.numpy as jnp
import numpy as np
from jax import lax
from jax.experimental import pallas as pl
from jax.experimental.pallas import tpu as pltpu
from jax.experimental.pallas import tpu_sc as plsc

_FIELD_DIMS = [100000] * 26
_F = 26
_D = 16
_T = 4
_B = 4096
_BF = _B * _F
_TOTAL = sum(_FIELD_DIMS)
_NLINES = _TOTAL // 8

_info = plsc.get_sparse_core_info()
_NC, _NS = _info.num_cores, _info.num_subcores
_NW = _NC * _NS
_NPW = _BF // _NW
_BW = _B // _NW

_NBLK = _TOTAL // 128 + 1
_NSUP = (_NBLK - 1) // 4
_SPW = (_NSUP + _NW - 1) // _NW
_NL = _NBLK * 16


@functools.partial(
    pl.kernel,
    out_type=jax.ShapeDtypeStruct((_NL, 128), jnp.float32),
    mesh=plsc.VectorSubcoreMesh(core_axis_name="c", subcore_axis_name="s"),
    compiler_params=pltpu.CompilerParams(needs_layout_passes=False),
    scratch_types=[
        pltpu.VMEM((_D, 512), jnp.float32),
        pltpu.VMEM((_D, 512), jnp.float32),
        pltpu.VMEM((64, 128), jnp.float32),
        pltpu.VMEM((64, 128), jnp.float32),
        pltpu.SemaphoreType.DMA,
        pltpu.SemaphoreType.DMA,
    ],
)
def _sc_retile(e2T_hbm, tail_hbm, out_hbm, bufA, bufB, outA, outB, sA, sB):
    wid = lax.axis_index("s") * _NC + lax.axis_index("c")
    base = wid * _SPW
    iota = lax.iota(jnp.int32, 16)
    last = _NSUP - 1

    rowv0 = iota >> 3
    colv = [((iota & 7) << 4) + d for d in range(_D)]

    def load(sb, buf, sem):
        return pltpu.async_copy(e2T_hbm.at[:, pl.ds(sb * 512, 512)], buf, sem)

    def emit(sb, buf, ob):
        for q in range(4):
            for c in range(8):
                rowv = rowv0 + (16 * q + 2 * c)
                vals = [buf[d, pl.ds(q * 128 + c * 16, 16)] for d in range(_D)]
                for d in range(_D):
                    plsc.store_scatter(ob, [rowv, colv[d]], vals[d])
        pltpu.sync_copy(ob, out_hbm.at[pl.ds(sb * 64, 64), :])

    def body(k, carry):
        sb0 = jnp.minimum(base + 2 * k, last)
        sb1 = jnp.minimum(base + 2 * k + 1, last)
        cA = load(sb0, bufA, sA)
        cB = load(sb1, bufB, sB)
        cA.wait()
        emit(sb0, bufA, outA)
        cB.wait()
        emit(sb1, bufB, outB)
        return carry

    lax.fori_loop(0, (_SPW + 1) // 2, body, 0)

    pltpu.sync_copy(tail_hbm, outB.at[pl.ds(0, 8), :])
    pltpu.sync_copy(outB.at[pl.ds(0, 8), :],
                    out_hbm.at[pl.ds((_TOTAL - 64) // 8, 8), :])


@functools.partial(
    pl.kernel,
    out_type=[
        jax.ShapeDtypeStruct((_F * _D, _B), jnp.float32),
        jax.ShapeDtypeStruct((_B,), jnp.float32),
    ],
    mesh=plsc.VectorSubcoreMesh(core_axis_name="c", subcore_axis_name="s"),
    compiler_params=pltpu.CompilerParams(needs_layout_passes=False),
    scratch_types=[
        pltpu.VMEM((_NPW,), jnp.int32),
        pltpu.VMEM((_NPW,), jnp.int32),
        pltpu.VMEM((_NPW,), jnp.int32),
        pltpu.VMEM((_NPW,), jnp.int32),
        pltpu.VMEM((_BW, 128), jnp.float32),
        pltpu.VMEM((_BW, 128), jnp.float32),
        pltpu.VMEM((_BW, 128), jnp.float32),
        pltpu.VMEM((_BW, 128), jnp.float32),
        pltpu.VMEM((_D, _BW), jnp.float32),
        pltpu.VMEM((_BW,), jnp.float32),
        pltpu.SemaphoreType.DMA,
        pltpu.SemaphoreType.DMA,
        pltpu.SemaphoreType.DMA,
        pltpu.SemaphoreType.DMA,
    ],
)
def _sc_gather(l2_hbm, c2_hbm, l1_hbm, c1_hbm, e2l_hbm, e1l_hbm, eT_out, lin_out,
               l2v, c2v, l1v, c1v, bufA2, bufB2, bufA1, bufB1, out_f, lin_acc,
               sA, sB, sA1, sB1):
    wid = lax.axis_index("s") * _NC + lax.axis_index("c")
    base = wid * _NPW
    bcol = wid * _BW
    iota = lax.iota(jnp.int32, 16)

    pltpu.sync_copy(l2_hbm.at[pl.ds(base, _NPW)], l2v)
    pltpu.sync_copy(c2_hbm.at[pl.ds(base, _NPW)], c2v)
    pltpu.sync_copy(l1_hbm.at[pl.ds(base, _NPW)], l1v)
    pltpu.sync_copy(c1_hbm.at[pl.ds(base, _NPW)], c1v)

    zero16 = jnp.zeros((16,), jnp.float32)
    for g in range(_BW // 16):
        lin_acc[pl.ds(g * 16, 16)] = zero16

    def start(f, b2, b1, s2, s1):
        c2 = pltpu.async_copy(e2l_hbm.at[l2v.at[pl.ds(f * _BW, _BW)]], b2, s2)
        c1 = pltpu.async_copy(e1l_hbm.at[l1v.at[pl.ds(f * _BW, _BW)]], b1, s1)
        return c2, c1

    def extract(f, b2, b1):
        for g in range(_BW // 16):
            rows = iota + g * 16
            sel = f * _BW + g * 16 + iota
            r2 = plsc.load_gather(c2v, [sel])
            r1 = plsc.load_gather(c1v, [sel])
            for d in range(_D):
                v = plsc.load_gather(b2, [rows, r2 + d])
                out_f[d, pl.ds(g * 16, 16)] = v
            v1 = plsc.load_gather(b1, [rows, r1])
            lin_acc[pl.ds(g * 16, 16)] = lin_acc[pl.ds(g * 16, 16)] + v1
        pltpu.sync_copy(out_f, eT_out.at[pl.ds(f * _D, _D), pl.ds(bcol, _BW)])

    def body(k, carry):
        f0 = 2 * k
        f1 = 2 * k + 1
        cA2, cA1 = start(f0, bufA2, bufA1, sA, sA1)
        cB2, cB1 = start(f1, bufB2, bufB1, sB, sB1)
        cA2.wait()
        cA1.wait()
        extract(f0, bufA2, bufA1)
        cB2.wait()
        cB1.wait()
        extract(f1, bufB2, bufB1)
        return carry

    lax.fori_loop(0, _F // 2, body, 0)
    pltpu.sync_copy(lin_acc, lin_out.at[pl.ds(bcol, _BW)])


_BT = 128
_PAIRS = _F * (_F - 1) // 2


def _tc_body(eT_ref, lin_ref, const_ref, out_ref):
    eT = eT_ref[...]
    C = const_ref[...]

    wts = [C[_D * t:_D * (t + 1), :] for t in range(_T + 1)]
    u_parts = [[] for _ in range(_T + 1)]
    for i in range(_F - 1):
        cnt = _F - 1 - i
        left = eT[_D * i:_D * (i + 1), :]
        right = eT[_D * (i + 1):, :]
        lrep = jnp.concatenate([left] * cnt, axis=0)
        pr3 = (lrep * right).reshape(cnt, _D, _BT)
        for t in range(_T + 1):
            u_parts[t].append(jnp.sum(pr3 * wts[t][None, :, :], axis=1))
    us = [jnp.concatenate(u_parts[t], axis=0) for t in range(_T + 1)]

    score = jnp.zeros((_PAIRS, _BT), jnp.float32)
    for t in range(_T):
        b1_t = C[80 + t:81 + t, :]
        w2_t = C[84 + t:85 + t, :]
        score = score + w2_t * jnp.maximum(us[t] + b1_t, 0.0)

    m = jnp.max(score, axis=0, keepdims=True)
    ex = jnp.exp(score - m)
    z = jnp.sum(ex, axis=0, keepdims=True)
    numer = jnp.sum(ex * us[_T], axis=0, keepdims=True)
    attr_part = numer / z

    lin = lin_ref[...]
    w0v = C[88:89, :]
    logit = w0v + lin + attr_part
    out = 1.0 / (1.0 + jnp.exp(-logit))
    out_ref[...] = jnp.broadcast_to(out, (8, _BT))


def _tc_compute(eT, lin2d, const):
    grid = _B // _BT
    return pl.pallas_call(
        _tc_body,
        grid=(grid,),
        in_specs=[
            pl.BlockSpec((_F * _D, _BT), lambda i: (0, i)),
            pl.BlockSpec((1, _BT), lambda i: (0, i)),
            pl.BlockSpec((96, _BT), lambda i: (0, 0)),
        ],
        out_specs=pl.BlockSpec((8, _BT), lambda i: (0, i)),
        out_shape=jax.ShapeDtypeStruct((8, _B), jnp.float32),
    )(eT, lin2d, const)


def kernel(x, emb1, emb2, w0, p, W1, b1, W2):
    offsets = jnp.asarray(np.cumsum([0] + _FIELD_DIMS[:-1]), dtype=x.dtype)
    idxm = x + offsets[None, :]
    idx_w = idxm.T.reshape(_F, _NW, _BW).transpose(1, 0, 2).reshape(_NW * _NPW)
    lines2 = idx_w // 8
    cols2 = (idx_w % 8) * _D
    lines1 = idx_w // 128
    cols1 = idx_w % 128

    tail_lines = emb2[_TOTAL - 64:].reshape(8, 128)
    e2lines = _sc_retile(emb2.T, tail_lines)
    _L1 = _TOTAL // 128 + 1
    e1lines = jnp.pad(emb1, ((0, _L1 * 128 - _TOTAL), (0, 0))).reshape(_L1, 128)

    eT, lin = _sc_gather(lines2, cols2, lines1, cols1, e2lines, e1lines)

    W5 = jnp.concatenate([W1, p[:, None]], axis=1)
    top = jnp.repeat(W5.T.reshape(5 * _D, 1), _BT, axis=1)
    sc9 = jnp.concatenate([b1, W2[:, 0], w0, jnp.zeros((7,), jnp.float32)])
    bot = jnp.repeat(sc9.reshape(16, 1), _BT, axis=1)
    const = jnp.concatenate([top, bot], axis=0)

    o8 = _tc_compute(eT, lin.reshape(1, _B), const)
    return o8[0].reshape(_B, 1)


# speedup vs baseline: 2.3182x; 1.0313x over previous
import functools

import jax
import jax.numpy as jnp
import numpy as np
from jax import lax
from jax.experimental import pallas as pl
from jax.experimental.pallas import tpu as pltpu
from jax.experimental.pallas import tpu_sc as plsc

_FIELD_DIMS = [100000] * 26
_F = 26
_D = 16
_T = 4
_B = 4096
_BF = _B * _F
_TOTAL = sum(_FIELD_DIMS)
_NLINES = _TOTAL // 8

_info = plsc.get_sparse_core_info()
_NC, _NS = _info.num_cores, _info.num_subcores
_NW = _NC * _NS
_NPW = _BF // _NW
_BW = _B // _NW

_NBLK = _TOTAL // 128 + 1
_NSUP = (_NBLK - 1) // 4
_SPW = (_NSUP + _NW - 1) // _NW
_NL = _NBLK * 16


@functools.partial(
    pl.kernel,
    out_type=jax.ShapeDtypeStruct((_NL, 128), jnp.float32),
    mesh=plsc.VectorSubcoreMesh(core_axis_name="c", subcore_axis_name="s"),
    compiler_params=pltpu.CompilerParams(needs_layout_passes=False),
    scratch_types=[
        pltpu.VMEM((_D, 512), jnp.float32),
        pltpu.VMEM((_D, 512), jnp.float32),
        pltpu.VMEM((64, 128), jnp.float32),
        pltpu.VMEM((64, 128), jnp.float32),
        pltpu.SemaphoreType.DMA,
        pltpu.SemaphoreType.DMA,
        pltpu.SemaphoreType.DMA,
        pltpu.SemaphoreType.DMA,
    ],
)
def _sc_retile(e2T_hbm, tail_hbm, out_hbm, bufA, bufB, outA, outB,
               sA, sB, sOA, sOB):
    wid = lax.axis_index("s") * _NC + lax.axis_index("c")
    base = wid * _SPW
    iota = lax.iota(jnp.int32, 16)
    last = _NSUP - 1

    rowv0 = iota >> 3
    colv = [((iota & 7) << 4) + d for d in range(_D)]

    def load(sb, buf, sem):
        return pltpu.async_copy(e2T_hbm.at[:, pl.ds(sb * 512, 512)], buf, sem)

    def emit(sb, buf, ob, osem):
        for q in range(4):
            for c in range(8):
                rowv = rowv0 + (16 * q + 2 * c)
                vals = [buf[d, pl.ds(q * 128 + c * 16, 16)] for d in range(_D)]
                for d in range(_D):
                    plsc.store_scatter(ob, [rowv, colv[d]], vals[d])
        return pltpu.async_copy(ob, out_hbm.at[pl.ds(sb * 64, 64), :], osem)

    def body(k, carry):
        sb0 = jnp.minimum(base + 2 * k, last)
        sb1 = jnp.minimum(base + 2 * k + 1, last)
        cA = load(sb0, bufA, sA)
        cB = load(sb1, bufB, sB)
        cA.wait()
        oA = emit(sb0, bufA, outA, sOA)
        cB.wait()
        oB = emit(sb1, bufB, outB, sOB)
        oA.wait()
        oB.wait()
        return carry

    lax.fori_loop(0, (_SPW + 1) // 2, body, 0)

    pltpu.sync_copy(tail_hbm, outB.at[pl.ds(0, 8), :])
    pltpu.sync_copy(outB.at[pl.ds(0, 8), :],
                    out_hbm.at[pl.ds((_TOTAL - 64) // 8, 8), :])


@functools.partial(
    pl.kernel,
    out_type=[
        jax.ShapeDtypeStruct((_F * _D, _B), jnp.float32),
        jax.ShapeDtypeStruct((_B,), jnp.float32),
    ],
    mesh=plsc.VectorSubcoreMesh(core_axis_name="c", subcore_axis_name="s"),
    compiler_params=pltpu.CompilerParams(needs_layout_passes=False),
    scratch_types=[
        pltpu.VMEM((_NPW,), jnp.int32),
        pltpu.VMEM((_NPW,), jnp.int32),
        pltpu.VMEM((_NPW,), jnp.int32),
        pltpu.VMEM((_NPW,), jnp.int32),
        pltpu.VMEM((_BW, 128), jnp.float32),
        pltpu.VMEM((_BW, 128), jnp.float32),
        pltpu.VMEM((_BW, 128), jnp.float32),
        pltpu.VMEM((_BW, 128), jnp.float32),
        pltpu.VMEM((_D, _BW), jnp.float32),
        pltpu.VMEM((_BW,), jnp.float32),
        pltpu.SemaphoreType.DMA,
        pltpu.SemaphoreType.DMA,
        pltpu.SemaphoreType.DMA,
        pltpu.SemaphoreType.DMA,
    ],
)
def _sc_gather(l2_hbm, c2_hbm, l1_hbm, c1_hbm, e2l_hbm, e1l_hbm, eT_out, lin_out,
               l2v, c2v, l1v, c1v, bufA2, bufB2, bufA1, bufB1, out_f, lin_acc,
               sA, sB, sA1, sB1):
    wid = lax.axis_index("s") * _NC + lax.axis_index("c")
    base = wid * _NPW
    bcol = wid * _BW
    iota = lax.iota(jnp.int32, 16)

    pltpu.sync_copy(l2_hbm.at[pl.ds(base, _NPW)], l2v)
    pltpu.sync_copy(c2_hbm.at[pl.ds(base, _NPW)], c2v)
    pltpu.sync_copy(l1_hbm.at[pl.ds(base, _NPW)], l1v)
    pltpu.sync_copy(c1_hbm.at[pl.ds(base, _NPW)], c1v)

    zero16 = jnp.zeros((16,), jnp.float32)
    for g in range(_BW // 16):
        lin_acc[pl.ds(g * 16, 16)] = zero16

    def start(f, b2, b1, s2, s1):
        c2 = pltpu.async_copy(e2l_hbm.at[l2v.at[pl.ds(f * _BW, _BW)]], b2, s2)
        c1 = pltpu.async_copy(e1l_hbm.at[l1v.at[pl.ds(f * _BW, _BW)]], b1, s1)
        return c2, c1

    def extract(f, b2, b1):
        for g in range(_BW // 16):
            rows = iota + g * 16
            sel = f * _BW + g * 16 + iota
            r2 = plsc.load_gather(c2v, [sel])
            r1 = plsc.load_gather(c1v, [sel])
            for d in range(_D):
                v = plsc.load_gather(b2, [rows, r2 + d])
                out_f[d, pl.ds(g * 16, 16)] = v
            v1 = plsc.load_gather(b1, [rows, r1])
            lin_acc[pl.ds(g * 16, 16)] = lin_acc[pl.ds(g * 16, 16)] + v1
        pltpu.sync_copy(out_f, eT_out.at[pl.ds(f * _D, _D), pl.ds(bcol, _BW)])

    def body(k, carry):
        f0 = 2 * k
        f1 = 2 * k + 1
        cA2, cA1 = start(f0, bufA2, bufA1, sA, sA1)
        cB2, cB1 = start(f1, bufB2, bufB1, sB, sB1)
        cA2.wait()
        cA1.wait()
        extract(f0, bufA2, bufA1)
        cB2.wait()
        cB1.wait()
        extract(f1, bufB2, bufB1)
        return carry

    lax.fori_loop(0, _F // 2, body, 0)
    pltpu.sync_copy(lin_acc, lin_out.at[pl.ds(bcol, _BW)])


_BT = 128
_PAIRS = _F * (_F - 1) // 2


def _tc_body(eT_ref, lin_ref, const_ref, out_ref):
    eT = eT_ref[...]
    C = const_ref[...]

    wts = [C[_D * t:_D * (t + 1), :] for t in range(_T + 1)]
    u_parts = [[] for _ in range(_T + 1)]
    for i in range(_F - 1):
        cnt = _F - 1 - i
        left = eT[_D * i:_D * (i + 1), :]
        right = eT[_D * (i + 1):, :]
        lrep = jnp.concatenate([left] * cnt, axis=0)
        pr3 = (lrep * right).reshape(cnt, _D, _BT)
        for t in range(_T + 1):
            u_parts[t].append(jnp.sum(pr3 * wts[t][None, :, :], axis=1))
    us = [jnp.concatenate(u_parts[t], axis=0) for t in range(_T + 1)]

    score = jnp.zeros((_PAIRS, _BT), jnp.float32)
    for t in range(_T):
        b1_t = C[80 + t:81 + t, :]
        w2_t = C[84 + t:85 + t, :]
        score = score + w2_t * jnp.maximum(us[t] + b1_t, 0.0)

    m = jnp.max(score, axis=0, keepdims=True)
    ex = jnp.exp(score - m)
    z = jnp.sum(ex, axis=0, keepdims=True)
    numer = jnp.sum(ex * us[_T], axis=0, keepdims=True)
    attr_part = numer / z

    lin = lin_ref[...]
    w0v = C[88:89, :]
    logit = w0v + lin + attr_part
    out = 1.0 / (1.0 + jnp.exp(-logit))
    out_ref[...] = jnp.broadcast_to(out, (8, _BT))


def _tc_compute(eT, lin2d, const):
    grid = _B // _BT
    return pl.pallas_call(
        _tc_body,
        grid=(grid,),
        in_specs=[
            pl.BlockSpec((_F * _D, _BT), lambda i: (0, i)),
            pl.BlockSpec((1, _BT), lambda i: (0, i)),
            pl.BlockSpec((96, _BT), lambda i: (0, 0)),
        ],
        out_specs=pl.BlockSpec((8, _BT), lambda i: (0, i)),
        out_shape=jax.ShapeDtypeStruct((8, _B), jnp.float32),
    )(eT, lin2d, const)


def kernel(x, emb1, emb2, w0, p, W1, b1, W2):
    offsets = jnp.asarray(np.cumsum([0] + _FIELD_DIMS[:-1]), dtype=x.dtype)
    idxm = x + offsets[None, :]
    idx_w = idxm.T.reshape(_F, _NW, _BW).transpose(1, 0, 2).reshape(_NW * _NPW)
    lines2 = idx_w // 8
    cols2 = (idx_w % 8) * _D
    lines1 = idx_w // 128
    cols1 = idx_w % 128

    tail_lines = emb2[_TOTAL - 64:].reshape(8, 128)
    e2lines = _sc_retile(emb2.T, tail_lines)
    _L1 = _TOTAL // 128 + 1
    e1lines = jnp.pad(emb1, ((0, _L1 * 128 - _TOTAL), (0, 0))).reshape(_L1, 128)

    eT, lin = _sc_gather(lines2, cols2, lines1, cols1, e2lines, e1lines)

    W5 = jnp.concatenate([W1, p[:, None]], axis=1)
    top = jnp.repeat(W5.T.reshape(5 * _D, 1), _BT, axis=1)
    sc9 = jnp.concatenate([b1, W2[:, 0], w0, jnp.zeros((7,), jnp.float32)])
    bot = jnp.repeat(sc9.reshape(16, 1), _BT, axis=1)
    const = jnp.concatenate([top, bot], axis=0)

    o8 = _tc_compute(eT, lin.reshape(1, _B), const)
    return o8[0].reshape(_B, 1)


# speedup vs baseline: 2.3849x; 1.0288x over previous
import functools

import jax
import jax.numpy as jnp
import numpy as np
from jax import lax
from jax.experimental import pallas as pl
from jax.experimental.pallas import tpu as pltpu
from jax.experimental.pallas import tpu_sc as plsc

_FIELD_DIMS = [100000] * 26
_F = 26
_D = 16
_T = 4
_B = 4096
_BF = _B * _F
_TOTAL = sum(_FIELD_DIMS)
_NLINES = _TOTAL // 8

_info = plsc.get_sparse_core_info()
_NC, _NS = _info.num_cores, _info.num_subcores
_NW = _NC * _NS
_NPW = _BF // _NW
_BW = _B // _NW

_NBLK = _TOTAL // 128 + 1
_NSUP = (_NBLK - 1) // 4
_SPW = (_NSUP + _NW - 1) // _NW
_NL = _NBLK * 16


@functools.partial(
    pl.kernel,
    out_type=jax.ShapeDtypeStruct((_NL, 128), jnp.float32),
    mesh=plsc.VectorSubcoreMesh(core_axis_name="c", subcore_axis_name="s"),
    compiler_params=pltpu.CompilerParams(needs_layout_passes=False),
    scratch_types=[
        pltpu.VMEM((_D, 512), jnp.float32),
        pltpu.VMEM((_D, 512), jnp.float32),
        pltpu.VMEM((64, 128), jnp.float32),
        pltpu.VMEM((64, 128), jnp.float32),
        pltpu.SemaphoreType.DMA,
        pltpu.SemaphoreType.DMA,
        pltpu.SemaphoreType.DMA,
        pltpu.SemaphoreType.DMA,
    ],
)
def _sc_retile(e2T_hbm, tail_hbm, out_hbm, bufA, bufB, outA, outB,
               sA, sB, sOA, sOB):
    wid = lax.axis_index("s") * _NC + lax.axis_index("c")
    base = wid * _SPW
    iota = lax.iota(jnp.int32, 16)
    last = _NSUP - 1

    rowv0 = iota >> 3
    colv = [((iota & 7) << 4) + d for d in range(_D)]

    def load(sb, buf, sem):
        return pltpu.async_copy(e2T_hbm.at[:, pl.ds(sb * 512, 512)], buf, sem)

    def emit(sb, buf, ob, osem):
        for q in range(4):
            for c in range(8):
                rowv = rowv0 + (16 * q + 2 * c)
                vals = [buf[d, pl.ds(q * 128 + c * 16, 16)] for d in range(_D)]
                for d in range(_D):
                    plsc.store_scatter(ob, [rowv, colv[d]], vals[d])
        return pltpu.async_copy(ob, out_hbm.at[pl.ds(sb * 64, 64), :], osem)

    def body(k, carry):
        sb0 = jnp.minimum(base + 2 * k, last)
        sb1 = jnp.minimum(base + 2 * k + 1, last)
        cA = load(sb0, bufA, sA)
        cB = load(sb1, bufB, sB)
        cA.wait()
        oA = emit(sb0, bufA, outA, sOA)
        cB.wait()
        oB = emit(sb1, bufB, outB, sOB)
        oA.wait()
        oB.wait()
        return carry

    lax.fori_loop(0, (_SPW + 1) // 2, body, 0)

    pltpu.sync_copy(tail_hbm, outB.at[pl.ds(0, 8), :])
    pltpu.sync_copy(outB.at[pl.ds(0, 8), :],
                    out_hbm.at[pl.ds((_TOTAL - 64) // 8, 8), :])


@functools.partial(
    pl.kernel,
    out_type=[
        jax.ShapeDtypeStruct((_F * _D, _B), jnp.float32),
        jax.ShapeDtypeStruct((_B,), jnp.float32),
    ],
    mesh=plsc.VectorSubcoreMesh(core_axis_name="c", subcore_axis_name="s"),
    compiler_params=pltpu.CompilerParams(needs_layout_passes=False),
    scratch_types=[
        pltpu.VMEM((_NPW,), jnp.int32),
        pltpu.VMEM((_NPW,), jnp.int32),
        pltpu.VMEM((_NPW,), jnp.int32),
        pltpu.VMEM((_NPW,), jnp.int32),
        pltpu.VMEM((_BW, 128), jnp.float32),
        pltpu.VMEM((_BW, 128), jnp.float32),
        pltpu.VMEM((_BW, 128), jnp.float32),
        pltpu.VMEM((_BW, 128), jnp.float32),
        pltpu.VMEM((_D, _BW), jnp.float32),
        pltpu.VMEM((_D, _BW), jnp.float32),
        pltpu.VMEM((_BW,), jnp.float32),
        pltpu.SemaphoreType.DMA,
        pltpu.SemaphoreType.DMA,
        pltpu.SemaphoreType.DMA,
        pltpu.SemaphoreType.DMA,
        pltpu.SemaphoreType.DMA,
        pltpu.SemaphoreType.DMA,
    ],
)
def _sc_gather(l2_hbm, c2_hbm, l1_hbm, c1_hbm, e2l_hbm, e1l_hbm, eT_out, lin_out,
               l2v, c2v, l1v, c1v, bufA2, bufB2, bufA1, bufB1, out_fA, out_fB,
               lin_acc, sA, sB, sA1, sB1, sOA, sOB):
    wid = lax.axis_index("s") * _NC + lax.axis_index("c")
    base = wid * _NPW
    bcol = wid * _BW
    iota = lax.iota(jnp.int32, 16)

    pltpu.sync_copy(l2_hbm.at[pl.ds(base, _NPW)], l2v)
    pltpu.sync_copy(c2_hbm.at[pl.ds(base, _NPW)], c2v)
    pltpu.sync_copy(l1_hbm.at[pl.ds(base, _NPW)], l1v)
    pltpu.sync_copy(c1_hbm.at[pl.ds(base, _NPW)], c1v)

    zero16 = jnp.zeros((16,), jnp.float32)
    for g in range(_BW // 16):
        lin_acc[pl.ds(g * 16, 16)] = zero16

    def start(f, b2, b1, s2, s1):
        c2 = pltpu.async_copy(e2l_hbm.at[l2v.at[pl.ds(f * _BW, _BW)]], b2, s2)
        c1 = pltpu.async_copy(e1l_hbm.at[l1v.at[pl.ds(f * _BW, _BW)]], b1, s1)
        return c2, c1

    def extract(f, b2, b1, of, osem):
        for g in range(_BW // 16):
            rows = iota + g * 16
            sel = f * _BW + g * 16 + iota
            r2 = plsc.load_gather(c2v, [sel])
            r1 = plsc.load_gather(c1v, [sel])
            vals = [plsc.load_gather(b2, [rows, r2 + d]) for d in range(_D)]
            for d in range(_D):
                of[d, pl.ds(g * 16, 16)] = vals[d]
            v1 = plsc.load_gather(b1, [rows, r1])
            lin_acc[pl.ds(g * 16, 16)] = lin_acc[pl.ds(g * 16, 16)] + v1
        return pltpu.async_copy(
            of, eT_out.at[pl.ds(f * _D, _D), pl.ds(bcol, _BW)], osem)

    def body(k, carry):
        f0 = 2 * k
        f1 = 2 * k + 1
        cA2, cA1 = start(f0, bufA2, bufA1, sA, sA1)
        cB2, cB1 = start(f1, bufB2, bufB1, sB, sB1)
        cA2.wait()
        cA1.wait()
        oA = extract(f0, bufA2, bufA1, out_fA, sOA)
        cB2.wait()
        cB1.wait()
        oB = extract(f1, bufB2, bufB1, out_fB, sOB)
        oA.wait()
        oB.wait()
        return carry

    lax.fori_loop(0, _F // 2, body, 0)
    pltpu.sync_copy(lin_acc, lin_out.at[pl.ds(bcol, _BW)])


_BT = 128
_PAIRS = _F * (_F - 1) // 2


def _tc_body(eT_ref, lin_ref, const_ref, out_ref):
    eT = eT_ref[...]
    C = const_ref[...]

    wts = [C[_D * t:_D * (t + 1), :] for t in range(_T + 1)]
    u_parts = [[] for _ in range(_T + 1)]
    for i in range(_F - 1):
        cnt = _F - 1 - i
        left = eT[_D * i:_D * (i + 1), :]
        right = eT[_D * (i + 1):, :]
        lrep = jnp.concatenate([left] * cnt, axis=0)
        pr3 = (lrep * right).reshape(cnt, _D, _BT)
        for t in range(_T + 1):
            u_parts[t].append(jnp.sum(pr3 * wts[t][None, :, :], axis=1))
    us = [jnp.concatenate(u_parts[t], axis=0) for t in range(_T + 1)]

    score = jnp.zeros((_PAIRS, _BT), jnp.float32)
    for t in range(_T):
        b1_t = C[80 + t:81 + t, :]
        w2_t = C[84 + t:85 + t, :]
        score = score + w2_t * jnp.maximum(us[t] + b1_t, 0.0)

    m = jnp.max(score, axis=0, keepdims=True)
    ex = jnp.exp(score - m)
    z = jnp.sum(ex, axis=0, keepdims=True)
    numer = jnp.sum(ex * us[_T], axis=0, keepdims=True)
    attr_part = numer / z

    lin = lin_ref[...]
    w0v = C[88:89, :]
    logit = w0v + lin + attr_part
    out = 1.0 / (1.0 + jnp.exp(-logit))
    out_ref[...] = jnp.broadcast_to(out, (8, _BT))


def _tc_compute(eT, lin2d, const):
    grid = _B // _BT
    return pl.pallas_call(
        _tc_body,
        grid=(grid,),
        in_specs=[
            pl.BlockSpec((_F * _D, _BT), lambda i: (0, i)),
            pl.BlockSpec((1, _BT), lambda i: (0, i)),
            pl.BlockSpec((96, _BT), lambda i: (0, 0)),
        ],
        out_specs=pl.BlockSpec((8, _BT), lambda i: (0, i)),
        out_shape=jax.ShapeDtypeStruct((8, _B), jnp.float32),
    )(eT, lin2d, const)


def kernel(x, emb1, emb2, w0, p, W1, b1, W2):
    offsets = jnp.asarray(np.cumsum([0] + _FIELD_DIMS[:-1]), dtype=x.dtype)
    idxm = x + offsets[None, :]
    idx_w = idxm.T.reshape(_F, _NW, _BW).transpose(1, 0, 2).reshape(_NW * _NPW)
    lines2 = idx_w // 8
    cols2 = (idx_w % 8) * _D
    lines1 = idx_w // 128
    cols1 = idx_w % 128

    tail_lines = emb2[_TOTAL - 64:].reshape(8, 128)
    e2lines = _sc_retile(emb2.T, tail_lines)
    _L1 = _TOTAL // 128 + 1
    e1lines = jnp.pad(emb1, ((0, _L1 * 128 - _TOTAL), (0, 0))).reshape(_L1, 128)

    eT, lin = _sc_gather(lines2, cols2, lines1, cols1, e2lines, e1lines)

    W5 = jnp.concatenate([W1, p[:, None]], axis=1)
    top = jnp.repeat(W5.T.reshape(5 * _D, 1), _BT, axis=1)
    sc9 = jnp.concatenate([b1, W2[:, 0], w0, jnp.zeros((7,), jnp.float32)])
    bot = jnp.repeat(sc9.reshape(16, 1), _BT, axis=1)
    const = jnp.concatenate([top, bot], axis=0)

    o8 = _tc_compute(eT, lin.reshape(1, _B), const)
    return o8[0].reshape(_B, 1)
